# baseline (speedup 1.0000x reference)
import functools

import jax
import jax.numpy as jnp
from jax import lax
from jax.experimental import pallas as pl
from jax.experimental.pallas import tpu as pltpu
from jax.experimental.pallas import tpu_sc as plsc

_DIM = 1024
_ROWS = 8192
_SCALE = _DIM ** (-0.5)

_NC = 2
_NS = 16
_L = 16
_NW = _NC * _NS

_ROWS_PER_W = _ROWS // _NW
_CHUNK_ROWS = 16
_NCHUNKS = _ROWS_PER_W // _CHUNK_ROWS
_VECS_PER_ROW = _DIM // _L

_mesh = plsc.VectorSubcoreMesh(core_axis_name="c", subcore_axis_name="s")


_NBUF = 7
_PF = 4


@functools.partial(
    pl.kernel,
    mesh=_mesh,
    out_type=jax.ShapeDtypeStruct((_ROWS, _DIM), jnp.float32),
    scratch_types=[
        pltpu.VMEM((_NBUF, _CHUNK_ROWS, _DIM), jnp.float32),
        pltpu.SemaphoreType.DMA((_NBUF,)),
        pltpu.SemaphoreType.DMA((_NBUF,)),
    ],
    compiler_params=pltpu.CompilerParams(
        use_tc_tiling_on_sc=True,
        disable_bounds_checks=True,
        disable_semaphore_checks=True,
    ),
)
def _scale_sc(emb_hbm, out_hbm, buf, in_sems, out_sems):
    wid = lax.axis_index("s") * _NC + lax.axis_index("c")
    row0 = wid * _ROWS_PER_W

    def in_copy(g, b):
        return pltpu.make_async_copy(
            emb_hbm.at[pl.ds(row0 + g * _CHUNK_ROWS, _CHUNK_ROWS)],
            buf.at[b], in_sems.at[b])

    def out_copy(g, b):
        return pltpu.make_async_copy(
            buf.at[b],
            out_hbm.at[pl.ds(row0 + g * _CHUNK_ROWS, _CHUNK_ROWS)],
            out_sems.at[b])

    for p in range(_PF):
        in_copy(p, p).start()

    def chunk_body(g, carry):
        b = g % _NBUF
        in_copy(g, b).wait()

        @pl.when(g + _PF < _NCHUNKS)
        def _prefetch():
            b2 = (g + _PF) % _NBUF

            @pl.when(g >= _NBUF - _PF)
            def _drain():
                out_copy(g + _PF - _NBUF, b2).wait()

            in_copy(g + _PF, b2).start()

        @plsc.parallel_loop(0, _CHUNK_ROWS, step=1, unroll=4)
        def scale_row(r):
            for c in range(_VECS_PER_ROW):
                sl = pl.ds(c * _L, _L)
                buf[b, r, sl] = buf[b, r, sl] * _SCALE

        out_copy(g, b).start()
        return carry

    lax.fori_loop(0, _NCHUNKS, chunk_body, 0)
    for g in range(_NCHUNKS - _NBUF, _NCHUNKS):
        if g >= 0:
            out_copy(g, g % _NBUF).wait()


def kernel(x, emb):
    del x
    return _scale_sc(emb)


# speedup vs baseline: 1.0256x; 1.0256x over previous
import functools

import jax
import jax.numpy as jnp
from jax import lax
from jax.experimental import pallas as pl
from jax.experimental.pallas import tpu as pltpu
from jax.experimental.pallas import tpu_sc as plsc

_DIM = 1024
_ROWS = 8192
_SCALE = _DIM ** (-0.5)

_NC = 2
_NS = 16
_L = 16
_NW = _NC * _NS

_ROWS_PER_W = _ROWS // _NW
_CHUNK_ROWS = 16
_NCHUNKS = _ROWS_PER_W // _CHUNK_ROWS
_VECS_PER_ROW = _DIM // _L

_mesh = plsc.VectorSubcoreMesh(core_axis_name="c", subcore_axis_name="s")


_NBUF = 7
_PF = _NBUF - 1


@functools.partial(
    pl.kernel,
    mesh=_mesh,
    out_type=jax.ShapeDtypeStruct((_ROWS, _DIM), jnp.float32),
    scratch_types=[
        pltpu.VMEM((_NBUF, _CHUNK_ROWS, _DIM), jnp.float32),
        pltpu.SemaphoreType.DMA((_NBUF,)),
        pltpu.SemaphoreType.DMA((_NBUF,)),
    ],
    compiler_params=pltpu.CompilerParams(
        use_tc_tiling_on_sc=True,
        disable_bounds_checks=True,
        disable_semaphore_checks=True,
    ),
)
def _scale_sc(emb_hbm, out_hbm, buf, in_sems, out_sems):
    wid = lax.axis_index("s") * _NC + lax.axis_index("c")
    row0 = wid * _ROWS_PER_W

    def in_copy(g, b):
        return pltpu.make_async_copy(
            emb_hbm.at[pl.ds(row0 + g * _CHUNK_ROWS, _CHUNK_ROWS)],
            buf.at[b], in_sems.at[b])

    def out_copy(g, b):
        return pltpu.make_async_copy(
            buf.at[b],
            out_hbm.at[pl.ds(row0 + g * _CHUNK_ROWS, _CHUNK_ROWS)],
            out_sems.at[b])

    for p in range(_PF):
        in_copy(p, p).start()

    def chunk_body(g, carry):
        b = g % _NBUF
        in_copy(g, b).wait()

        @pl.when(g + _PF < _NCHUNKS)
        def _prefetch():
            b2 = (g + _PF) % _NBUF

            @pl.when(g >= _NBUF - _PF)
            def _drain():
                out_copy(g + _PF - _NBUF, b2).wait()

            in_copy(g + _PF, b2).start()

        @plsc.parallel_loop(0, _CHUNK_ROWS, step=1, unroll=4)
        def scale_row(r):
            for c in range(_VECS_PER_ROW):
                sl = pl.ds(c * _L, _L)
                buf[b, r, sl] = buf[b, r, sl] * _SCALE

        out_copy(g, b).start()
        return carry

    lax.fori_loop(0, _NCHUNKS, chunk_body, 0)
    for g in range(_NCHUNKS - _NBUF, _NCHUNKS):
        if g >= 0:
            out_copy(g, g % _NBUF).wait()


def kernel(x, emb):
    del x
    return _scale_sc(emb)
